# baseline (speedup 1.0000x reference)
import functools

import jax
import jax.numpy as jnp
from jax.experimental import pallas as pl
from jax.experimental.pallas import tpu as pltpu
from jax.experimental.pallas import tpu_sc as plsc

B, S, D, E, F = 32, 4, 1024, 8, 2048
T = B * S
LANES = 16
NCH = T // LANES


def _logits_kernel(x_ref, rw_ref, rb_ref, lg_ref, lgt_ref):
    lg = jnp.dot(x_ref[...], rw_ref[...],
                 preferred_element_type=jnp.float32) + rb_ref[...]
    lg_ref[...] = lg
    lgt_ref[...] = lg.T


def _router_sc_body(logits_hbm, scale_hbm, stats_hbm, logits_v, scale_v,
                    stats_v):
    cid = jax.lax.axis_index("c")
    sid = jax.lax.axis_index("s")

    @pl.when((cid == 0) & (sid == 0))
    def _():
        pltpu.sync_copy(logits_hbm, logits_v)
        acc_p = [None] * E
        acc_c = [None] * E
        for ch in range(NCH):
            sl = pl.ds(ch * LANES, LANES)
            l = [logits_v[e, sl] for e in range(E)]
            m = l[0]
            for e in range(1, E):
                m = jnp.maximum(m, l[e])
            ex = [jnp.exp(l[e] - m) for e in range(E)]
            den = ex[0]
            for e in range(1, E):
                den = den + ex[e]
            inv = 1.0 / den
            p = [ex[e] * inv for e in range(E)]
            pmax = p[0]
            for e in range(1, E):
                pmax = jnp.maximum(pmax, p[e])
            found = jnp.zeros((LANES,), jnp.float32)
            for e in range(E):
                ge = jnp.where(l[e] >= m, 1.0, 0.0)
                fe = ge * (1.0 - found)
                scale_v[e, sl] = fe * pmax
                found = found + fe
                acc_c[e] = fe if ch == 0 else acc_c[e] + fe
                acc_p[e] = p[e] if ch == 0 else acc_p[e] + p[e]
        for e in range(E):
            stats_v[e, :] = acc_c[e]
            stats_v[E + e, :] = acc_p[e]
        pltpu.sync_copy(scale_v, scale_hbm)
        pltpu.sync_copy(stats_v, stats_hbm)


_router_sc = functools.partial(
    pl.kernel,
    out_type=[jax.ShapeDtypeStruct((E, T), jnp.float32),
              jax.ShapeDtypeStruct((2 * E, LANES), jnp.float32)],
    mesh=plsc.VectorSubcoreMesh(core_axis_name="c", subcore_axis_name="s"),
    scratch_types=[pltpu.VMEM((E, T), jnp.float32),
                   pltpu.VMEM((E, T), jnp.float32),
                   pltpu.VMEM((2 * E, LANES), jnp.float32)],
)(_router_sc_body)


def _ffn_kernel(x_ref, st_ref, stats_ref,
                w1a_ref, w1b_ref, w1c_ref, w1d_ref, b1_ref,
                w2a_ref, w2b_ref, w2c_ref, w2d_ref, b2_ref,
                out_ref, loss_ref, scale_ref):
    e = pl.program_id(0)

    @pl.when(e == 0)
    def _scale_transpose():
        scale_ref[...] = st_ref[...].T
        stats = stats_ref[...]
        cnt = jnp.sum(stats[:E, :], axis=1, keepdims=True)
        sump = jnp.sum(stats[E:, :], axis=1, keepdims=True)
        loss = jnp.sum(cnt * sump) * (E / (T * T))
        loss_ref[...] = loss.reshape(1, 1)

    xx = x_ref[...].astype(jnp.bfloat16)
    dq, fq = D // 4, F // 4
    w1refs = (w1a_ref, w1b_ref, w1c_ref, w1d_ref)
    h = sum(jnp.dot(xx[:, i * dq:(i + 1) * dq],
                    w1refs[i][0].astype(jnp.bfloat16),
                    preferred_element_type=jnp.float32) for i in range(4))
    h = jnp.maximum(h + b1_ref[0], 0.0).astype(jnp.bfloat16)
    w2refs = (w2a_ref, w2b_ref, w2c_ref, w2d_ref)
    part = sum(jnp.dot(h[:, i * fq:(i + 1) * fq],
                       w2refs[i][0].astype(jnp.bfloat16),
                       preferred_element_type=jnp.float32) for i in range(4))

    lane = jax.lax.broadcasted_iota(jnp.int32, (T, E), 1)
    s = jnp.sum(scale_ref[...] * (lane == e).astype(jnp.float32),
                axis=1, keepdims=True)
    contrib = s * (part + b2_ref[0])

    @pl.when(e == 0)
    def _first():
        out_ref[...] = contrib

    @pl.when(e > 0)
    def _rest():
        out_ref[...] += contrib


@functools.partial(jax.jit, static_argnames=("interpret",))
def _moe(x, router_w, router_b, w1, b1, w2, b2, interpret=False):
    x_flat = x.reshape(T, D)
    rb = router_b.reshape(1, E)
    b1r = b1.reshape(E, 1, F)
    b2r = b2.reshape(E, 1, D)

    logits, logits_t = pl.pallas_call(
        _logits_kernel,
        in_specs=[pl.BlockSpec((T, D), lambda: (0, 0)),
                  pl.BlockSpec((D, E), lambda: (0, 0)),
                  pl.BlockSpec((1, E), lambda: (0, 0))],
        out_specs=[pl.BlockSpec((T, E), lambda: (0, 0)),
                   pl.BlockSpec((E, T), lambda: (0, 0))],
        out_shape=[jax.ShapeDtypeStruct((T, E), jnp.float32),
                   jax.ShapeDtypeStruct((E, T), jnp.float32)],
        interpret=interpret,
    )(x_flat, router_w, rb)

    scale_t, stats = _router_sc(logits_t)

    out, loss = pl.pallas_call(
        _ffn_kernel,
        grid=(E,),
        in_specs=[
            pl.BlockSpec((T, D), lambda e: (0, 0)),
            pl.BlockSpec((E, T), lambda e: (0, 0)),
            pl.BlockSpec((2 * E, LANES), lambda e: (0, 0)),
            *[pl.BlockSpec((1, D // 4, F), (lambda i: lambda e: (e, i, 0))(i))
              for i in range(4)],
            pl.BlockSpec((1, 1, F), lambda e: (e, 0, 0)),
            *[pl.BlockSpec((1, F // 4, D), (lambda i: lambda e: (e, i, 0))(i))
              for i in range(4)],
            pl.BlockSpec((1, 1, D), lambda e: (e, 0, 0)),
        ],
        out_specs=[pl.BlockSpec((T, D), lambda e: (0, 0)),
                   pl.BlockSpec((1, 1), lambda e: (0, 0))],
        out_shape=[jax.ShapeDtypeStruct((T, D), jnp.float32),
                   jax.ShapeDtypeStruct((1, 1), jnp.float32)],
        scratch_shapes=[pltpu.VMEM((T, E), jnp.float32)],
        compiler_params=pltpu.CompilerParams(
            dimension_semantics=("arbitrary",)),
        interpret=interpret,
    )(x_flat, scale_t, stats, w1, w1, w1, w1, b1r, w2, w2, w2, w2, b2r)
    return out.reshape(B, S, D), loss[0, 0], logits


def kernel(x, router_w, router_b, w1, b1, w2, b2):
    return _moe(x, router_w, router_b, w1, b1, w2, b2)


# speedup vs baseline: 1.2999x; 1.2999x over previous
import functools

import jax
import jax.numpy as jnp
from jax.experimental import pallas as pl
from jax.experimental.pallas import tpu as pltpu

B, S, D, E, F = 32, 4, 1024, 8, 2048
T = B * S


def _moe_kernel(x_ref, rw_ref, rb_ref,
                w1a_ref, w1b_ref, w1c_ref, w1d_ref, b1_ref,
                w2a_ref, w2b_ref, w2c_ref, w2d_ref, b2_ref,
                out_ref, loss_ref, logits_ref, h_ref, scale_ref):
    e = pl.program_id(0)
    j = pl.program_id(1)

    @pl.when((e == 0) & (j == 0))
    def _router():
        xx = x_ref[...]
        logits = jnp.dot(xx, rw_ref[...],
                         preferred_element_type=jnp.float32) + rb_ref[...]
        logits_ref[...] = logits
        m = jnp.max(logits, axis=-1, keepdims=True)
        ex = jnp.exp(logits - m)
        probs = ex / jnp.sum(ex, axis=-1, keepdims=True)
        pmax = jnp.max(probs, axis=-1, keepdims=True)
        lane = jax.lax.broadcasted_iota(jnp.int32, (T, E), 1)
        first = jnp.min(jnp.where(probs == pmax, lane, E), axis=-1,
                        keepdims=True)
        onehot = (lane == first).astype(jnp.float32)
        scale_ref[...] = onehot * pmax
        f_frac = jnp.sum(onehot, axis=0) * (1.0 / T)
        p_mean = jnp.sum(probs, axis=0) * (1.0 / T)
        loss_ref[...] = (E * jnp.sum(f_frac * p_mean)).reshape(1, 1)

    dq, fq = D // 4, F // 4

    @pl.when(j == 0)
    def _first_matmul():
        xx = x_ref[...].astype(jnp.bfloat16)
        w1refs = (w1a_ref, w1b_ref, w1c_ref, w1d_ref)
        h = sum(jnp.dot(xx[:, i * dq:(i + 1) * dq],
                        w1refs[i][0].astype(jnp.bfloat16),
                        preferred_element_type=jnp.float32)
                for i in range(4))
        h_ref[...] = jnp.maximum(h + b1_ref[0], 0.0).astype(jnp.bfloat16)

    @pl.when(j == 1)
    def _second_matmul():
        hh = h_ref[...]
        w2refs = (w2a_ref, w2b_ref, w2c_ref, w2d_ref)
        part = sum(jnp.dot(hh[:, i * fq:(i + 1) * fq],
                           w2refs[i][0].astype(jnp.bfloat16),
                           preferred_element_type=jnp.float32)
                   for i in range(4))

        lane = jax.lax.broadcasted_iota(jnp.int32, (T, E), 1)
        s = jnp.sum(scale_ref[...] * (lane == e).astype(jnp.float32),
                    axis=1, keepdims=True)
        contrib = s * (part + b2_ref[0])

        @pl.when(e == 0)
        def _first():
            out_ref[...] = contrib

        @pl.when(e > 0)
        def _rest():
            out_ref[...] += contrib


@functools.partial(jax.jit, static_argnames=("interpret",))
def _moe(x, router_w, router_b, w1, b1, w2, b2, interpret=False):
    x_flat = x.reshape(T, D)
    rb = router_b.reshape(1, E)
    b1r = b1.reshape(E, 1, F)
    b2r = b2.reshape(E, 1, D)
    out, loss, logits = pl.pallas_call(
        _moe_kernel,
        grid=(E, 2),
        in_specs=[
            pl.BlockSpec((T, D), lambda e, j: (0, 0)),
            pl.BlockSpec((D, E), lambda e, j: (0, 0)),
            pl.BlockSpec((1, E), lambda e, j: (0, 0)),
            *[pl.BlockSpec((1, D // 4, F),
                           (lambda i: lambda e, j: (e, i, 0))(i))
              for i in range(4)],
            pl.BlockSpec((1, 1, F), lambda e, j: (e, 0, 0)),
            *[pl.BlockSpec((1, F // 4, D),
                           (lambda i: lambda e, j:
                            (jnp.maximum(e + j - 1, 0), i, 0))(i))
              for i in range(4)],
            pl.BlockSpec((1, 1, D), lambda e, j: (e, 0, 0)),
        ],
        out_specs=[
            pl.BlockSpec((T, D), lambda e, j: (0, 0)),
            pl.BlockSpec((1, 1), lambda e, j: (0, 0)),
            pl.BlockSpec((T, E), lambda e, j: (0, 0)),
        ],
        out_shape=[
            jax.ShapeDtypeStruct((T, D), jnp.float32),
            jax.ShapeDtypeStruct((1, 1), jnp.float32),
            jax.ShapeDtypeStruct((T, E), jnp.float32),
        ],
        scratch_shapes=[
            pltpu.VMEM((T, F), jnp.bfloat16),
            pltpu.VMEM((T, E), jnp.float32),
        ],
        compiler_params=pltpu.CompilerParams(
            dimension_semantics=("arbitrary", "arbitrary")),
        interpret=interpret,
    )(x_flat, router_w, rb, w1, w1, w1, w1, b1r, w2, w2, w2, w2, b2r)
    return out.reshape(B, S, D), loss[0, 0], logits


def kernel(x, router_w, router_b, w1, b1, w2, b2):
    return _moe(x, router_w, router_b, w1, b1, w2, b2)


# speedup vs baseline: 1.3567x; 1.0437x over previous
import functools

import jax
import jax.numpy as jnp
from jax.experimental import pallas as pl
from jax.experimental.pallas import tpu as pltpu

B, S, D, E, F = 32, 4, 1024, 8, 2048
T = B * S


def _moe_kernel(x_ref, rw_ref, rb_ref, w1_ref, b1_ref, w2_ref, b2_ref,
                out_ref, loss_ref, logits_ref,
                w1buf, w2buf, scale_ref, w1sem, w2sem):
    def w1cp(e):
        return pltpu.make_async_copy(w1_ref.at[e], w1buf.at[e % 2],
                                     w1sem.at[e % 2])

    def w2cp(e):
        return pltpu.make_async_copy(w2_ref.at[e], w2buf.at[e % 2],
                                     w2sem.at[e % 2])

    for e in range(2):
        w1cp(e).start()
        w2cp(e).start()

    xx = x_ref[...]
    logits = jnp.dot(xx, rw_ref[...],
                     preferred_element_type=jnp.float32) + rb_ref[...]
    logits_ref[...] = logits
    m = jnp.max(logits, axis=-1, keepdims=True)
    ex = jnp.exp(logits - m)
    probs = ex / jnp.sum(ex, axis=-1, keepdims=True)
    pmax = jnp.max(probs, axis=-1, keepdims=True)
    lane = jax.lax.broadcasted_iota(jnp.int32, (T, E), 1)
    first = jnp.min(jnp.where(probs == pmax, lane, E), axis=-1,
                    keepdims=True)
    onehot = (lane == first).astype(jnp.float32)
    scale = onehot * pmax
    scale_ref[...] = scale
    f_frac = jnp.sum(onehot, axis=0) * (1.0 / T)
    p_mean = jnp.sum(probs, axis=0) * (1.0 / T)
    loss_ref[...] = (E * jnp.sum(f_frac * p_mean)).reshape(1, 1)

    xb = xx.astype(jnp.bfloat16)
    for e in range(E):
        w1cp(e).wait()
        h = jnp.dot(xb, w1buf[e % 2].astype(jnp.bfloat16),
                    preferred_element_type=jnp.float32)
        h = jnp.maximum(h + b1_ref[e], 0.0).astype(jnp.bfloat16)
        w2cp(e).wait()
        part = jnp.dot(h, w2buf[e % 2].astype(jnp.bfloat16),
                       preferred_element_type=jnp.float32)
        if e + 2 < E:
            w1cp(e + 2).start()
            w2cp(e + 2).start()
        s = scale_ref[:, e:e + 1]
        contrib = s * (part + b2_ref[e])
        if e == 0:
            out_ref[...] = contrib
        else:
            out_ref[...] += contrib


@functools.partial(jax.jit, static_argnames=("interpret",))
def _moe(x, router_w, router_b, w1, b1, w2, b2, interpret=False):
    x_flat = x.reshape(T, D)
    rb = router_b.reshape(1, E)
    out, loss, logits = pl.pallas_call(
        _moe_kernel,
        in_specs=[
            pl.BlockSpec((T, D), lambda: (0, 0)),
            pl.BlockSpec((D, E), lambda: (0, 0)),
            pl.BlockSpec((1, E), lambda: (0, 0)),
            pl.BlockSpec(memory_space=pltpu.MemorySpace.HBM),
            pl.BlockSpec((E, F), lambda: (0, 0)),
            pl.BlockSpec(memory_space=pltpu.MemorySpace.HBM),
            pl.BlockSpec((E, D), lambda: (0, 0)),
        ],
        out_specs=[
            pl.BlockSpec((T, D), lambda: (0, 0)),
            pl.BlockSpec((1, 1), lambda: (0, 0)),
            pl.BlockSpec((T, E), lambda: (0, 0)),
        ],
        out_shape=[
            jax.ShapeDtypeStruct((T, D), jnp.float32),
            jax.ShapeDtypeStruct((1, 1), jnp.float32),
            jax.ShapeDtypeStruct((T, E), jnp.float32),
        ],
        scratch_shapes=[
            pltpu.VMEM((2, D, F), jnp.float32),
            pltpu.VMEM((2, F, D), jnp.float32),
            pltpu.VMEM((T, E), jnp.float32),
            pltpu.SemaphoreType.DMA((2,)),
            pltpu.SemaphoreType.DMA((2,)),
        ],
        interpret=interpret,
    )(x_flat, router_w, rb, w1, b1, w2, b2)
    return out.reshape(B, S, D), loss[0, 0], logits


def kernel(x, router_w, router_b, w1, b1, w2, b2):
    return _moe(x, router_w, router_b, w1, b1, w2, b2)


# speedup vs baseline: 1.3600x; 1.0024x over previous
import functools

import jax
import jax.numpy as jnp
from jax.experimental import pallas as pl
from jax.experimental.pallas import tpu as pltpu

B, S, D, E, F = 32, 4, 1024, 8, 2048
T = B * S


def _moe_kernel(x_ref, rw_ref, rb_ref, w1_ref, b1_ref, w2_ref, b2_ref,
                out_ref, loss_ref, logits_ref,
                w1buf, w2buf, scale_ref, w1sem, w2sem):
    def w1cp(e):
        return pltpu.make_async_copy(w1_ref.at[e], w1buf.at[e % 3],
                                     w1sem.at[e % 3])

    def w2cp(e):
        return pltpu.make_async_copy(w2_ref.at[e], w2buf.at[e % 3],
                                     w2sem.at[e % 3])

    for e in range(3):
        w1cp(e).start()
        w2cp(e).start()

    xx = x_ref[...]
    logits = jnp.dot(xx, rw_ref[...],
                     preferred_element_type=jnp.float32) + rb_ref[...]
    logits_ref[...] = logits
    m = jnp.max(logits, axis=-1, keepdims=True)
    ex = jnp.exp(logits - m)
    probs = ex / jnp.sum(ex, axis=-1, keepdims=True)
    pmax = jnp.max(probs, axis=-1, keepdims=True)
    lane = jax.lax.broadcasted_iota(jnp.int32, (T, E), 1)
    first = jnp.min(jnp.where(probs == pmax, lane, E), axis=-1,
                    keepdims=True)
    onehot = (lane == first).astype(jnp.float32)
    scale = onehot * pmax
    scale_ref[...] = scale
    f_frac = jnp.sum(onehot, axis=0) * (1.0 / T)
    p_mean = jnp.sum(probs, axis=0) * (1.0 / T)
    loss_ref[...] = (E * jnp.sum(f_frac * p_mean)).reshape(1, 1)

    xb = xx.astype(jnp.bfloat16)
    for e in range(E):
        w1cp(e).wait()
        h = jnp.dot(xb, w1buf[e % 3].astype(jnp.bfloat16),
                    preferred_element_type=jnp.float32)
        h = jnp.maximum(h + b1_ref[e], 0.0).astype(jnp.bfloat16)
        w2cp(e).wait()
        part = jnp.dot(h, w2buf[e % 3].astype(jnp.bfloat16),
                       preferred_element_type=jnp.float32)
        if e + 3 < E:
            w1cp(e + 3).start()
            w2cp(e + 3).start()
        s = scale_ref[:, e:e + 1]
        contrib = s * (part + b2_ref[e])
        if e == 0:
            out_ref[...] = contrib
        else:
            out_ref[...] += contrib


@functools.partial(jax.jit, static_argnames=("interpret",))
def _moe(x, router_w, router_b, w1, b1, w2, b2, interpret=False):
    x_flat = x.reshape(T, D)
    rb = router_b.reshape(1, E)
    out, loss, logits = pl.pallas_call(
        _moe_kernel,
        in_specs=[
            pl.BlockSpec((T, D), lambda: (0, 0)),
            pl.BlockSpec((D, E), lambda: (0, 0)),
            pl.BlockSpec((1, E), lambda: (0, 0)),
            pl.BlockSpec(memory_space=pltpu.MemorySpace.HBM),
            pl.BlockSpec((E, F), lambda: (0, 0)),
            pl.BlockSpec(memory_space=pltpu.MemorySpace.HBM),
            pl.BlockSpec((E, D), lambda: (0, 0)),
        ],
        out_specs=[
            pl.BlockSpec((T, D), lambda: (0, 0)),
            pl.BlockSpec((1, 1), lambda: (0, 0)),
            pl.BlockSpec((T, E), lambda: (0, 0)),
        ],
        out_shape=[
            jax.ShapeDtypeStruct((T, D), jnp.float32),
            jax.ShapeDtypeStruct((1, 1), jnp.float32),
            jax.ShapeDtypeStruct((T, E), jnp.float32),
        ],
        scratch_shapes=[
            pltpu.VMEM((3, D, F), jnp.float32),
            pltpu.VMEM((3, F, D), jnp.float32),
            pltpu.VMEM((T, E), jnp.float32),
            pltpu.SemaphoreType.DMA((3,)),
            pltpu.SemaphoreType.DMA((3,)),
        ],
        interpret=interpret,
    )(x_flat, router_w, rb, w1, b1, w2, b2)
    return out.reshape(B, S, D), loss[0, 0], logits


def kernel(x, router_w, router_b, w1, b1, w2, b2):
    return _moe(x, router_w, router_b, w1, b1, w2, b2)


# speedup vs baseline: 1.3672x; 1.0053x over previous
import functools

import jax
import jax.numpy as jnp
from jax.experimental import pallas as pl
from jax.experimental.pallas import tpu as pltpu

B, S, D, E, F = 32, 4, 1024, 8, 2048
T = B * S


def _moe_kernel(x_ref, rw_ref, rb_ref, w1_ref, b1_ref, w2_ref, b2_ref,
                out_ref, loss_ref, logits_ref,
                w1buf, w2buf, scale_ref, w1sem, w2sem):
    def w1cp(e):
        return pltpu.make_async_copy(w1_ref.at[e], w1buf.at[e % 3],
                                     w1sem.at[e % 3])

    def w2cp(e, half):
        sl = pl.ds(half * (F // 2), F // 2)
        return pltpu.make_async_copy(w2_ref.at[e, sl], w2buf.at[e % 3, sl],
                                     w2sem.at[e % 3, half])

    for e in range(3):
        w1cp(e).start()
        w2cp(e, 0).start()
        w2cp(e, 1).start()

    xx = x_ref[...]
    logits = jnp.dot(xx, rw_ref[...],
                     preferred_element_type=jnp.float32) + rb_ref[...]
    logits_ref[...] = logits
    m = jnp.max(logits, axis=-1, keepdims=True)
    ex = jnp.exp(logits - m)
    probs = ex / jnp.sum(ex, axis=-1, keepdims=True)
    pmax = jnp.max(probs, axis=-1, keepdims=True)
    lane = jax.lax.broadcasted_iota(jnp.int32, (T, E), 1)
    first = jnp.min(jnp.where(probs == pmax, lane, E), axis=-1,
                    keepdims=True)
    onehot = (lane == first).astype(jnp.float32)
    scale = onehot * pmax
    scale_ref[...] = scale
    f_frac = jnp.sum(onehot, axis=0) * (1.0 / T)
    p_mean = jnp.sum(probs, axis=0) * (1.0 / T)
    loss_ref[...] = (E * jnp.sum(f_frac * p_mean)).reshape(1, 1)

    xb = xx.astype(jnp.bfloat16)
    for e in range(E):
        w1cp(e).wait()
        h = jnp.dot(xb, w1buf[e % 3].astype(jnp.bfloat16),
                    preferred_element_type=jnp.float32)
        h = jnp.maximum(h + b1_ref[e], 0.0).astype(jnp.bfloat16)
        w2cp(e, 0).wait()
        part = jnp.dot(h[:, :F // 2], w2buf[e % 3, :F // 2].astype(jnp.bfloat16),
                       preferred_element_type=jnp.float32)
        w2cp(e, 1).wait()
        part += jnp.dot(h[:, F // 2:], w2buf[e % 3, F // 2:].astype(jnp.bfloat16),
                        preferred_element_type=jnp.float32)
        if e + 3 < E:
            w1cp(e + 3).start()
            w2cp(e + 3, 0).start()
            w2cp(e + 3, 1).start()
        s = scale_ref[:, e:e + 1]
        contrib = s * (part + b2_ref[e])
        if e == 0:
            out_ref[...] = contrib
        else:
            out_ref[...] += contrib


@functools.partial(jax.jit, static_argnames=("interpret",))
def _moe(x, router_w, router_b, w1, b1, w2, b2, interpret=False):
    x_flat = x.reshape(T, D)
    rb = router_b.reshape(1, E)
    out, loss, logits = pl.pallas_call(
        _moe_kernel,
        in_specs=[
            pl.BlockSpec((T, D), lambda: (0, 0)),
            pl.BlockSpec((D, E), lambda: (0, 0)),
            pl.BlockSpec((1, E), lambda: (0, 0)),
            pl.BlockSpec(memory_space=pltpu.MemorySpace.HBM),
            pl.BlockSpec((E, F), lambda: (0, 0)),
            pl.BlockSpec(memory_space=pltpu.MemorySpace.HBM),
            pl.BlockSpec((E, D), lambda: (0, 0)),
        ],
        out_specs=[
            pl.BlockSpec((T, D), lambda: (0, 0)),
            pl.BlockSpec((1, 1), lambda: (0, 0)),
            pl.BlockSpec((T, E), lambda: (0, 0)),
        ],
        out_shape=[
            jax.ShapeDtypeStruct((T, D), jnp.float32),
            jax.ShapeDtypeStruct((1, 1), jnp.float32),
            jax.ShapeDtypeStruct((T, E), jnp.float32),
        ],
        scratch_shapes=[
            pltpu.VMEM((3, D, F), jnp.float32),
            pltpu.VMEM((3, F, D), jnp.float32),
            pltpu.VMEM((T, E), jnp.float32),
            pltpu.SemaphoreType.DMA((3,)),
            pltpu.SemaphoreType.DMA((3, 2)),
        ],
        interpret=interpret,
    )(x_flat, router_w, rb, w1, b1, w2, b2)
    return out.reshape(B, S, D), loss[0, 0], logits


def kernel(x, router_w, router_b, w1, b1, w2, b2):
    return _moe(x, router_w, router_b, w1, b1, w2, b2)


# speedup vs baseline: 1.3765x; 1.0068x over previous
import functools

import jax
import jax.numpy as jnp
from jax.experimental import pallas as pl
from jax.experimental.pallas import tpu as pltpu

B, S, D, E, F = 32, 4, 1024, 8, 2048
T = B * S


def _moe_kernel(x_ref, rw_ref, rb_ref, w1_ref, b1_ref, w2_ref, b2_ref,
                out_ref, loss_ref, logits_ref,
                w1buf, w2buf, scale_ref, w1sem, w2sem):
    def w1cp(e, half):
        sl = pl.ds(half * (D // 2), D // 2)
        return pltpu.make_async_copy(w1_ref.at[e, sl], w1buf.at[e % 3, sl],
                                     w1sem.at[e % 3, half])

    def w2cp(e, q):
        sl = pl.ds(q * (F // 4), F // 4)
        return pltpu.make_async_copy(w2_ref.at[e, sl], w2buf.at[e % 3, sl],
                                     w2sem.at[e % 3, q])

    def start_expert(e):
        w1cp(e, 0).start()
        w1cp(e, 1).start()
        for q in range(4):
            w2cp(e, q).start()

    for e in range(3):
        start_expert(e)

    xx = x_ref[...]
    logits = jnp.dot(xx, rw_ref[...],
                     preferred_element_type=jnp.float32) + rb_ref[...]
    logits_ref[...] = logits
    m = jnp.max(logits, axis=-1, keepdims=True)
    ex = jnp.exp(logits - m)
    probs = ex / jnp.sum(ex, axis=-1, keepdims=True)
    pmax = jnp.max(probs, axis=-1, keepdims=True)
    lane = jax.lax.broadcasted_iota(jnp.int32, (T, E), 1)
    first = jnp.min(jnp.where(probs == pmax, lane, E), axis=-1,
                    keepdims=True)
    onehot = (lane == first).astype(jnp.float32)
    scale = onehot * pmax
    scale_ref[...] = scale
    f_frac = jnp.sum(onehot, axis=0) * (1.0 / T)
    p_mean = jnp.sum(probs, axis=0) * (1.0 / T)
    loss_ref[...] = (E * jnp.sum(f_frac * p_mean)).reshape(1, 1)

    xb = xx.astype(jnp.bfloat16)
    dh, fq = D // 2, F // 4
    for e in range(E):
        w1cp(e, 0).wait()
        hacc = jnp.dot(xb[:, :dh], w1buf[e % 3, :dh].astype(jnp.bfloat16),
                       preferred_element_type=jnp.float32)
        w1cp(e, 1).wait()
        hacc += jnp.dot(xb[:, dh:], w1buf[e % 3, dh:].astype(jnp.bfloat16),
                        preferred_element_type=jnp.float32)
        h = jnp.maximum(hacc + b1_ref[e], 0.0).astype(jnp.bfloat16)
        part = None
        for q in range(4):
            w2cp(e, q).wait()
            pq = jnp.dot(h[:, q * fq:(q + 1) * fq],
                         w2buf[e % 3, q * fq:(q + 1) * fq].astype(jnp.bfloat16),
                         preferred_element_type=jnp.float32)
            part = pq if part is None else part + pq
        if e + 3 < E:
            start_expert(e + 3)
        s = scale_ref[:, e:e + 1]
        contrib = s * (part + b2_ref[e])
        if e == 0:
            out_ref[...] = contrib
        else:
            out_ref[...] += contrib


@functools.partial(jax.jit, static_argnames=("interpret",))
def _moe(x, router_w, router_b, w1, b1, w2, b2, interpret=False):
    x_flat = x.reshape(T, D)
    rb = router_b.reshape(1, E)
    out, loss, logits = pl.pallas_call(
        _moe_kernel,
        in_specs=[
            pl.BlockSpec((T, D), lambda: (0, 0)),
            pl.BlockSpec((D, E), lambda: (0, 0)),
            pl.BlockSpec((1, E), lambda: (0, 0)),
            pl.BlockSpec(memory_space=pltpu.MemorySpace.HBM),
            pl.BlockSpec((E, F), lambda: (0, 0)),
            pl.BlockSpec(memory_space=pltpu.MemorySpace.HBM),
            pl.BlockSpec((E, D), lambda: (0, 0)),
        ],
        out_specs=[
            pl.BlockSpec((T, D), lambda: (0, 0)),
            pl.BlockSpec((1, 1), lambda: (0, 0)),
            pl.BlockSpec((T, E), lambda: (0, 0)),
        ],
        out_shape=[
            jax.ShapeDtypeStruct((T, D), jnp.float32),
            jax.ShapeDtypeStruct((1, 1), jnp.float32),
            jax.ShapeDtypeStruct((T, E), jnp.float32),
        ],
        scratch_shapes=[
            pltpu.VMEM((3, D, F), jnp.float32),
            pltpu.VMEM((3, F, D), jnp.float32),
            pltpu.VMEM((T, E), jnp.float32),
            pltpu.SemaphoreType.DMA((3, 2)),
            pltpu.SemaphoreType.DMA((3, 4)),
        ],
        interpret=interpret,
    )(x_flat, router_w, rb, w1, b1, w2, b2)
    return out.reshape(B, S, D), loss[0, 0], logits


def kernel(x, router_w, router_b, w1, b1, w2, b2):
    return _moe(x, router_w, router_b, w1, b1, w2, b2)
